# baseline (device time: 49115 ns/iter reference)
import jax
import jax.numpy as jnp
from jax import lax
from jax.experimental import pallas as pl
from jax.experimental.pallas import tpu as pltpu

N_DEV = 4


def kernel(dy, W):
    m, k = dy.shape
    n = W.shape[0]

    def body(dy_ref, w_ref, out_ref, comm_ref, send_sems, recv_sems):
        my = lax.axis_index("i")
        left = lax.rem(my + N_DEV - 1, N_DEV)
        right = lax.rem(my + 1, N_DEV)

        barrier_sem = pltpu.get_barrier_semaphore()
        for nbr in (left, right):
            pl.semaphore_signal(
                barrier_sem, inc=1,
                device_id=(nbr,), device_id_type=pl.DeviceIdType.MESH,
            )
        pl.semaphore_wait(barrier_sem, 2)

        partial = lax.dot_general(
            dy_ref[...].astype(jnp.bfloat16),
            w_ref[...].astype(jnp.bfloat16),
            dimension_numbers=(((1,), (1,)), ((), ())),
            preferred_element_type=jnp.float32,
        )
        out_ref[...] = partial
        comm_ref[0] = partial

        for h in range(N_DEV - 1):
            rdma = pltpu.make_async_remote_copy(
                src_ref=comm_ref.at[h],
                dst_ref=comm_ref.at[h + 1],
                send_sem=send_sems.at[h],
                recv_sem=recv_sems.at[h],
                device_id=(right,),
                device_id_type=pl.DeviceIdType.MESH,
            )
            rdma.start()
            rdma.wait()
            out_ref[...] += comm_ref[h + 1]

    return pl.pallas_call(
        body,
        out_shape=jax.ShapeDtypeStruct((m, n), jnp.float32),
        in_specs=[
            pl.BlockSpec(memory_space=pltpu.VMEM),
            pl.BlockSpec(memory_space=pltpu.VMEM),
        ],
        out_specs=pl.BlockSpec(memory_space=pltpu.VMEM),
        scratch_shapes=[
            pltpu.VMEM((N_DEV, m, n), jnp.float32),
            pltpu.SemaphoreType.DMA((N_DEV - 1,)),
            pltpu.SemaphoreType.DMA((N_DEV - 1,)),
        ],
        compiler_params=pltpu.CompilerParams(collective_id=0),
    )(dy, W)


# device time: 18373 ns/iter; 2.6732x vs baseline; 2.6732x over previous
import jax
import jax.numpy as jnp
from jax import lax
from jax.experimental import pallas as pl
from jax.experimental.pallas import tpu as pltpu

N_DEV = 4


def kernel(dy, W):
    m, k = dy.shape
    n = W.shape[0]
    q = m // N_DEV

    def body(dy_ref, w_ref, out_ref, part_ref, rs_buf, ag_send, ag_buf,
             rs_send_sems, rs_recv_sems, ag_send_sems, ag_recv_sems):
        my = lax.axis_index("i")

        barrier_sem = pltpu.get_barrier_semaphore()
        for o in range(1, N_DEV):
            pl.semaphore_signal(
                barrier_sem, inc=1,
                device_id=(lax.rem(my + o, N_DEV),),
                device_id_type=pl.DeviceIdType.MESH,
            )

        pf32 = lax.dot_general(
            dy_ref[...].astype(jnp.bfloat16),
            w_ref[...].astype(jnp.bfloat16),
            dimension_numbers=(((1,), (1,)), ((), ())),
            preferred_element_type=jnp.float32,
        )
        part_ref[...] = pf32.astype(jnp.bfloat16)

        pl.semaphore_wait(barrier_sem, N_DEV - 1)

        rs = []
        for o in (2, 1, 3):
            t = lax.rem(my + o, N_DEV)
            rdma = pltpu.make_async_remote_copy(
                src_ref=part_ref.at[pl.ds(t * q, q), :],
                dst_ref=rs_buf.at[3 - o],
                send_sem=rs_send_sems.at[o - 1],
                recv_sem=rs_recv_sems.at[3 - o],
                device_id=(t,),
                device_id_type=pl.DeviceIdType.MESH,
            )
            rdma.start()
            rs.append(rdma)

        out_ref[...] = pf32

        for rdma in rs:
            rdma.wait_recv()

        qs = pl.ds(my * q, q)
        acc = (out_ref[qs, :]
               + rs_buf[0].astype(jnp.float32)
               + rs_buf[1].astype(jnp.float32)
               + rs_buf[2].astype(jnp.float32))
        out_ref[qs, :] = acc
        ag_send[...] = acc.astype(jnp.bfloat16)

        ag = []
        for o in (2, 1, 3):
            t = lax.rem(my + o, N_DEV)
            rdma = pltpu.make_async_remote_copy(
                src_ref=ag_send,
                dst_ref=ag_buf.at[3 - o],
                send_sem=ag_send_sems.at[o - 1],
                recv_sem=ag_recv_sems.at[3 - o],
                device_id=(t,),
                device_id_type=pl.DeviceIdType.MESH,
            )
            rdma.start()
            ag.append(rdma)

        for s, rdma in zip((1, 2, 0), ag):
            rdma.wait_recv()
            origin = lax.rem(my + s + 1, N_DEV)
            out_ref[pl.ds(origin * q, q), :] = ag_buf[s].astype(jnp.float32)

        for rdma in rs + ag:
            rdma.wait_send()

    return pl.pallas_call(
        body,
        out_shape=jax.ShapeDtypeStruct((m, n), jnp.float32),
        in_specs=[
            pl.BlockSpec(memory_space=pltpu.VMEM),
            pl.BlockSpec(memory_space=pltpu.VMEM),
        ],
        out_specs=pl.BlockSpec(memory_space=pltpu.VMEM),
        scratch_shapes=[
            pltpu.VMEM((m, n), jnp.bfloat16),
            pltpu.VMEM((N_DEV - 1, q, n), jnp.bfloat16),
            pltpu.VMEM((q, n), jnp.bfloat16),
            pltpu.VMEM((N_DEV - 1, q, n), jnp.bfloat16),
            pltpu.SemaphoreType.DMA((N_DEV - 1,)),
            pltpu.SemaphoreType.DMA((N_DEV - 1,)),
            pltpu.SemaphoreType.DMA((N_DEV - 1,)),
            pltpu.SemaphoreType.DMA((N_DEV - 1,)),
        ],
        compiler_params=pltpu.CompilerParams(collective_id=0),
    )(dy, W)
